# Initial kernel scaffold; baseline (speedup 1.0000x reference)
import jax
import jax.numpy as jnp
from jax.experimental import pallas as pl


def kernel(att_s, att_t, v_s, v_t):
    raise NotImplementedError("write your pallas kernel here")


# speedup vs baseline: 4.9774x; 4.9774x over previous
import functools

import jax
import jax.numpy as jnp
from jax.experimental import pallas as pl

NUM_K = 10
FRAME_T = 8
SEG = 196


def _seg_out(x, v):
    b = jax.lax.bitcast_convert_type(x, jnp.int32)
    k = b ^ (jax.lax.shift_right_arithmetic(b, 31) & jnp.int32(0x7FFFFFFF))
    ktr = k & jnp.int32(-256)
    lane = jax.lax.broadcasted_iota(jnp.int32, x.shape, 1)
    kk = ktr | lane
    cur = kk
    thr = None
    for i in range(NUM_K):
        m = jnp.max(cur, axis=1, keepdims=True)
        if i == NUM_K - 1:
            thr = m
        else:
            cur = jnp.where(cur == m, jnp.int32(-2147483648), cur)
    mask = ktr >= (thr & jnp.int32(-256))
    xm = jnp.max(x, axis=1, keepdims=True)
    w = jnp.where(mask, jnp.exp(x - xm), 0.0)
    s = jnp.sum(w, axis=1, keepdims=True)
    o = jnp.dot(w, v, preferred_element_type=jnp.float32)
    return o / s


def _body(as_ref, at_ref, vs_ref, vt_ref, out_ref):
    @pl.when(jnp.logical_and(pl.program_id(0) == 0, pl.program_id(1) == 0))
    def _():
        out_ref[...] = jnp.zeros((1, 1), jnp.float32)

    acc = 0.0
    for t in range(FRAME_T):
        xs = as_ref[0, 0, :, t * SEG:(t + 1) * SEG]
        xt = at_ref[0, 0, :, t * SEG:(t + 1) * SEG]
        os_ = _seg_out(xs, vs_ref[0, t])
        ot_ = _seg_out(xt, vt_ref[0, t])
        d = os_ - ot_
        acc = acc + jnp.sum(d * d)
    out_ref[...] = out_ref[...] + acc


@jax.jit
def kernel(att_s, att_t, v_s, v_t):
    B, H, P, _ = att_t.shape
    h_dim = v_t.shape[-1]
    bp = 392
    np_ = P // bp
    v_rs = v_s.reshape(H, SEG, FRAME_T, h_dim).transpose(0, 2, 1, 3)
    v_rt = v_t.reshape(H, SEG, FRAME_T, h_dim).transpose(0, 2, 1, 3)

    total = pl.pallas_call(
        _body,
        grid=(H, np_),
        in_specs=[
            pl.BlockSpec((1, 1, bp, P), lambda h, p: (0, h, p, 0)),
            pl.BlockSpec((1, 1, bp, P), lambda h, p: (0, h, p, 0)),
            pl.BlockSpec((1, FRAME_T, SEG, h_dim), lambda h, p: (h, 0, 0, 0)),
            pl.BlockSpec((1, FRAME_T, SEG, h_dim), lambda h, p: (h, 0, 0, 0)),
        ],
        out_specs=pl.BlockSpec((1, 1), lambda h, p: (0, 0)),
        out_shape=jax.ShapeDtypeStruct((1, 1), jnp.float32),
    )(att_s, att_t, v_rs, v_rt)

    count = H * P * FRAME_T * h_dim
    return total[0, 0] / count


# speedup vs baseline: 8.4376x; 1.6952x over previous
import jax
import jax.numpy as jnp
from jax import lax
from jax.experimental import pallas as pl
from jax.experimental.pallas import tpu as pltpu
from jax.experimental.pallas import tpu_sc as plsc

NUM_K = 10
FRAME_T = 8
SEG = 196
NH = 12
P = 1568
HD = 64
CH = 28
NSUP = 14
HALF = 784
NW = 32
UPW = 3
WSEG = 2 * SEG


def _merge16(a, b):
    ka, ia = a
    kb, ib = b
    rk = jnp.flip(kb)
    ri = jnp.flip(ib)
    m = ka >= rk
    nk = jnp.where(m, ka, rk)
    ni = jnp.where(m, ia, ri)
    return plsc.sort_key_val(nk, ni, descending=True)


def _sel_topk(buf, r, seg):
    iota = lax.iota(jnp.int32, 16)
    base = seg * SEG
    items = []
    for j in range(13):
        off = j * 16 if j < 12 else 180
        k = buf[r, pl.ds(base + off, 16)]
        idx = iota + off
        if j == 12:
            k = jnp.where(iota >= 12, k, jnp.float32(-3.4e38))
        items.append(plsc.sort_key_val(k, idx, descending=True))
    while len(items) > 1:
        nxt = [_merge16(items[i], items[i + 1])
               for i in range(0, len(items) - 1, 2)]
        if len(items) % 2:
            nxt.append(items[-1])
        items = nxt
    return items[0]


def _row_out(buf, r, seg, vref):
    ck, ci = _sel_topk(buf, r, seg)
    iota = lax.iota(jnp.int32, 16)
    mx = jnp.max(ck)
    w = jnp.where(iota < NUM_K, jnp.exp(ck - mx), jnp.float32(0.0))
    swv = jnp.broadcast_to(jnp.sum(w), (16,))
    inv = jnp.ones((16,), jnp.float32) / swv
    accs = [jnp.zeros((16,), jnp.float32) for _ in range(4)]
    for i in range(NUM_K):
        di = ci[i]
        wi = w[i]
        for c in range(4):
            accs[c] = accs[c] + wi * vref[di, pl.ds(c * 16, 16)]
    return [a * inv for a in accs]


def _body(as_hbm, at_hbm, vs_hbm, vt_hbm, out_hbm,
          bs0, bs1, bt0, bt1, vs0, vs1, vt0, vt1,
          sqv, sem_s0, sem_s1, sem_t0, sem_t1):
    wid = lax.axis_index("s") * 2 + lax.axis_index("c")

    def unit_body(u, sq):
        unit = wid * UPW + u
        h = unit // 8
        rem = unit % 8
        tp = rem // 2
        p0 = (rem % 2) * HALF
        col0 = tp * WSEG

        pltpu.sync_copy(vs_hbm.at[h, 2 * tp], vs0)
        pltpu.sync_copy(vs_hbm.at[h, 2 * tp + 1], vs1)
        pltpu.sync_copy(vt_hbm.at[h, 2 * tp], vt0)
        pltpu.sync_copy(vt_hbm.at[h, 2 * tp + 1], vt1)

        def mk(chunk, hbm, buf, sem):
            src = hbm.at[h, pl.ds(p0 + chunk * CH, CH), pl.ds(col0, WSEG)]
            return pltpu.make_async_copy(src, buf, sem)

        def chunk_compute(bs, bt, sq):
            def row_body(r, sq):
                o_s0 = _row_out(bs, r, 0, vs0)
                o_t0 = _row_out(bt, r, 0, vt0)
                o_s1 = _row_out(bs, r, 1, vs1)
                o_t1 = _row_out(bt, r, 1, vt1)
                for c in range(4):
                    d0 = o_s0[c] - o_t0[c]
                    d1 = o_s1[c] - o_t1[c]
                    sq = sq + d0 * d0 + d1 * d1
                return sq
            return lax.fori_loop(0, CH, row_body, sq)

        mk(0, as_hbm, bs0, sem_s0).start()
        mk(0, at_hbm, bt0, sem_t0).start()

        def super_body(g, sq):
            mk(2 * g + 1, as_hbm, bs1, sem_s1).start()
            mk(2 * g + 1, at_hbm, bt1, sem_t1).start()
            mk(2 * g, as_hbm, bs0, sem_s0).wait()
            mk(2 * g, at_hbm, bt0, sem_t0).wait()
            sq = chunk_compute(bs0, bt0, sq)

            @pl.when(g < NSUP - 1)
            def _():
                mk(2 * g + 2, as_hbm, bs0, sem_s0).start()
                mk(2 * g + 2, at_hbm, bt0, sem_t0).start()

            mk(2 * g + 1, as_hbm, bs1, sem_s1).wait()
            mk(2 * g + 1, at_hbm, bt1, sem_t1).wait()
            return chunk_compute(bs1, bt1, sq)

        return lax.fori_loop(0, NSUP, super_body, sq)

    sq = lax.fori_loop(0, UPW, unit_body, jnp.zeros((16,), jnp.float32))
    sqv[...] = sq
    pltpu.sync_copy(sqv, out_hbm.at[wid])


_sc_call = pl.kernel(
    _body,
    out_type=jax.ShapeDtypeStruct((NW, 16), jnp.float32),
    mesh=plsc.VectorSubcoreMesh(core_axis_name="c", subcore_axis_name="s"),
    compiler_params=pltpu.CompilerParams(
        use_tc_tiling_on_sc=False, needs_layout_passes=False),
    scratch_types=[
        pltpu.VMEM((CH, WSEG), jnp.float32),
        pltpu.VMEM((CH, WSEG), jnp.float32),
        pltpu.VMEM((CH, WSEG), jnp.float32),
        pltpu.VMEM((CH, WSEG), jnp.float32),
        pltpu.VMEM((SEG, HD), jnp.float32),
        pltpu.VMEM((SEG, HD), jnp.float32),
        pltpu.VMEM((SEG, HD), jnp.float32),
        pltpu.VMEM((SEG, HD), jnp.float32),
        pltpu.VMEM((16,), jnp.float32),
        pltpu.SemaphoreType.DMA,
        pltpu.SemaphoreType.DMA,
        pltpu.SemaphoreType.DMA,
        pltpu.SemaphoreType.DMA,
    ],
)


@jax.jit
def kernel(att_s, att_t, v_s, v_t):
    as3 = att_s.reshape(NH, P, P)
    at3 = att_t.reshape(NH, P, P)
    v_rs = v_s.reshape(NH, SEG, FRAME_T, HD).transpose(0, 2, 1, 3)
    v_rt = v_t.reshape(NH, SEG, FRAME_T, HD).transpose(0, 2, 1, 3)
    out = _sc_call(as3, at3, v_rs, v_rt)
    return jnp.sum(out) / (NH * P * FRAME_T * HD)
